# Initial kernel scaffold; baseline (speedup 1.0000x reference)
import jax
import jax.numpy as jnp
from jax.experimental import pallas as pl


def kernel(atom_feats, edge_index, bond_feats, batch, atom_emb, bond_emb, eps, conv_W1, conv_b1, conv_W2, conv_b2, bn_gamma, bn_beta, ro_W1, ro_b1, ro_bn_gamma, ro_bn_beta, ro_W2, ro_b2):
    raise NotImplementedError("write your pallas kernel here")


# speedup vs baseline: 3.1838x; 3.1838x over previous
import functools

import jax
import jax.numpy as jnp
from jax import lax
from jax.experimental import pallas as pl
from jax.experimental.pallas import tpu as pltpu
import jax.experimental.pallas.tpu_sc as plsc

N, E, D, NG, NLAYERS, T = 50000, 800000, 64, 512, 4, 12
H = 32
NC, NS = 2, 16
NP = 51200
EP = 802816
NPB = NP // 128
EPB = EP // 128
NG_TILE = 25
EG = 256
EG_TILE = 196
NSTRIPE = NP // NS
BT = 1024
NBLK = NP // BT
DUMP = N

@functools.lru_cache(maxsize=None)
def _sc_mesh():
    return plsc.VectorSubcoreMesh(core_axis_name="c", subcore_axis_name="s",
                                  num_cores=NC, num_subcores=NS)


def _encode_body(tab_ref, aidx_ref, nrow_ref, out_ref, idxb, idxn, tmp, acc,
                 sem):
    c = lax.axis_index("c")
    s = lax.axis_index("s")

    @pl.loop(0, NG_TILE)
    def _grp(k):
        g = s * NG_TILE + k
        pltpu.sync_copy(nrow_ref.at[g], idxn)
        for f in range(9):
            pltpu.sync_copy(aidx_ref.at[(c * 9 + f) * NPB + g], idxb)
            pltpu.async_copy(tab_ref.at[idxb], tmp, sem).wait()
            pltpu.sync_copy(tmp, acc.at[idxn], add=(f > 0))

    pltpu.sync_copy(acc.at[pl.ds(s * NSTRIPE, NSTRIPE)],
                    out_ref.at[pl.ds(c * NP + s * NSTRIPE, NSTRIPE)])


@functools.lru_cache(maxsize=None)
def _encode():
    return pl.kernel(
        _encode_body,
        out_type=jax.ShapeDtypeStruct((NC * NP, H), jnp.float32),
        mesh=_sc_mesh(),
        scratch_types=[
            pltpu.VMEM((128,), jnp.int32),
            pltpu.VMEM((128,), jnp.int32),
            pltpu.VMEM((128, H), jnp.float32),
            pltpu.MemorySpace.VMEM_SHARED((NP, H), jnp.float32),
            pltpu.SemaphoreType.DMA,
        ],
        compiler_params=pltpu.CompilerParams(use_tc_tiling_on_sc=False),
    )


def _layer_body(x_ref, ct_ref, src_ref, dst_ref, cid_ref, out_ref,
                ib_s, ib_d0, ib_d1, ib_c, gx, ctb, acc, sem1, sem2):
    c = lax.axis_index("c")
    s = lax.axis_index("s")
    zero = jnp.zeros((16,), jnp.float32)

    @pl.loop(0, EG)
    def _zb(r):
        gx[r, pl.ds(0, 16)] = zero
        gx[r, pl.ds(16, 16)] = zero

    for q in range(12):
        pltpu.sync_copy(gx, acc.at[pl.ds(s * NSTRIPE + q * EG, EG)])
    pltpu.sync_copy(gx.at[pl.ds(0, 128)],
                    acc.at[pl.ds(s * NSTRIPE + 12 * EG, 128)])
    plsc.subcore_barrier()

    @pl.loop(0, EG_TILE)
    def _grp(g):
        r2 = (s * EG_TILE + g) * 2
        pltpu.sync_copy(src_ref.at[pl.ds(c * EPB + r2, 2)], ib_s)
        pltpu.sync_copy(cid_ref.at[pl.ds(c * EPB + r2, 2)], ib_c)
        pltpu.sync_copy(dst_ref.at[r2], ib_d0)
        pltpu.sync_copy(dst_ref.at[r2 + 1], ib_d1)
        descs = []
        for j in range(2):
            descs.append(pltpu.async_copy(
                x_ref.at[ib_s.at[j]], gx.at[pl.ds(j * 128, 128)], sem1))
            descs.append(pltpu.async_copy(
                ct_ref.at[ib_c.at[j]], ctb.at[pl.ds(j * 128, 128)], sem2))
        for dsc in descs:
            dsc.wait()

        @pl.loop(0, EG, unroll=4)
        def _msg(r):
            s0, s1 = pl.ds(0, 16), pl.ds(16, 16)
            gx[r, s0] = jnp.maximum(gx[r, s0] + ctb[r, s0], 0.0)
            gx[r, s1] = jnp.maximum(gx[r, s1] + ctb[r, s1], 0.0)

        pltpu.sync_copy(gx.at[pl.ds(0, 128)], acc.at[ib_d0], add=True)
        pltpu.sync_copy(gx.at[pl.ds(128, 128)], acc.at[ib_d1], add=True)

    plsc.subcore_barrier()
    pltpu.sync_copy(acc.at[pl.ds(s * NSTRIPE, NSTRIPE)],
                    out_ref.at[pl.ds(c * NP + s * NSTRIPE, NSTRIPE)])


@functools.lru_cache(maxsize=None)
def _layer_sc():
    return pl.kernel(
        _layer_body,
        out_type=jax.ShapeDtypeStruct((NC * NP, H), jnp.float32),
        mesh=_sc_mesh(),
        scratch_types=[
            pltpu.VMEM((2, 128), jnp.int32),
            pltpu.VMEM((128,), jnp.int32),
            pltpu.VMEM((128,), jnp.int32),
            pltpu.VMEM((2, 128), jnp.int32),
            pltpu.VMEM((EG, H), jnp.float32),
            pltpu.VMEM((EG, H), jnp.float32),
            pltpu.MemorySpace.VMEM_SHARED((NP, H), jnp.float32),
            pltpu.SemaphoreType.DMA,
            pltpu.SemaphoreType.DMA,
        ],
        compiler_params=pltpu.CompilerParams(use_tc_tiling_on_sc=False),
    )


def _combo_body(be_ref, out_ref):
    b0 = jnp.broadcast_to(be_ref[0][:, None, :], (8, 64, D)).reshape(NG, D)
    b1 = jnp.broadcast_to(be_ref[1][None, :, None, :],
                          (8, 8, 8, D)).reshape(NG, D)
    b2 = jnp.broadcast_to(be_ref[2][None, None, :, :],
                          (8, 8, 8, D)).reshape(NG, D)
    ct = (b0 + b1) + b2
    out_ref[0] = ct[:, :H]
    out_ref[1] = ct[:, H:]


_combo = pl.pallas_call(
    _combo_body,
    out_shape=jax.ShapeDtypeStruct((2, NG, H), jnp.float32),
)


def _mlp_body(eps_ref, x_ref, ag_ref, w1_ref, b1_ref, w2_ref, b2_ref,
              h2_ref, s_ref):
    i = pl.program_id(0)
    e = eps_ref[0]
    dot = functools.partial(lax.dot_general,
                            dimension_numbers=(((1,), (0,)), ((), ())),
                            preferred_element_type=jnp.float32)
    h0 = jnp.concatenate([x_ref[0] * e + ag_ref[0],
                          x_ref[1] * e + ag_ref[1]], axis=1)
    h1 = jnp.maximum(dot(h0, w1_ref[...]) + b1_ref[0:1, :], 0.0)
    h2 = jnp.maximum(dot(h1, w2_ref[...]) + b2_ref[0:1, :], 0.0)
    h2_ref[...] = h2
    rows = i * BT + lax.broadcasted_iota(jnp.int32, (BT, 1), 0)
    h2m = jnp.where(rows < N, h2, 0.0)
    ps = jnp.sum(h2m, axis=0, keepdims=True)

    @pl.when(i == 0)
    def _():
        s_ref[...] = jnp.zeros_like(s_ref)

    s_ref[...] += jnp.broadcast_to(ps, (8, D))


_mlp = pl.pallas_call(
    _mlp_body,
    grid=(NBLK,),
    in_specs=[
        pl.BlockSpec(memory_space=pltpu.SMEM),
        pl.BlockSpec((2, BT, H), lambda i: (0, i, 0)),
        pl.BlockSpec((2, BT, H), lambda i: (0, i, 0)),
        pl.BlockSpec((D, D), lambda i: (0, 0)),
        pl.BlockSpec((8, D), lambda i: (0, 0)),
        pl.BlockSpec((D, D), lambda i: (0, 0)),
        pl.BlockSpec((8, D), lambda i: (0, 0)),
    ],
    out_specs=[
        pl.BlockSpec((BT, D), lambda i: (i, 0)),
        pl.BlockSpec((8, D), lambda i: (0, 0)),
    ],
    out_shape=[
        jax.ShapeDtypeStruct((NP, D), jnp.float32),
        jax.ShapeDtypeStruct((8, D), jnp.float32),
    ],
    compiler_params=pltpu.CompilerParams(
        dimension_semantics=("arbitrary",)),
)


def _bnres_body(h2_ref, x_ref, s_ref, g_ref, b_ref, out_ref, acc):
    i = pl.program_id(0)
    j = i % NBLK
    mean = s_ref[0:1, :] / jnp.float32(N)

    @pl.when(i == 0)
    def _():
        acc[...] = jnp.zeros_like(acc)

    @pl.when(i < NBLK)
    def _():
        rows = j * BT + lax.broadcasted_iota(jnp.int32, (BT, 1), 0)
        dev = h2_ref[...] - mean
        dm = jnp.where(rows < N, dev * dev, 0.0)
        acc[...] += jnp.broadcast_to(jnp.sum(dm, axis=0, keepdims=True),
                                     (8, D))

    @pl.when(i >= NBLK)
    def _():
        var = acc[0:1, :] / jnp.float32(N)
        hn = jnp.maximum(
            g_ref[0:1, :] * (h2_ref[...] - mean) / jnp.sqrt(var + 1e-5)
            + b_ref[0:1, :], 0.0)
        out_ref[0] = hn[:, :H] + x_ref[0]
        out_ref[1] = hn[:, H:] + x_ref[1]


_bnres = pl.pallas_call(
    _bnres_body,
    grid=(2 * NBLK,),
    in_specs=[
        pl.BlockSpec((BT, D), lambda i: (i % NBLK, 0)),
        pl.BlockSpec((2, BT, H), lambda i: (0, i % NBLK, 0)),
        pl.BlockSpec((8, D), lambda i: (0, 0)),
        pl.BlockSpec((8, D), lambda i: (0, 0)),
        pl.BlockSpec((8, D), lambda i: (0, 0)),
    ],
    out_specs=pl.BlockSpec((2, BT, H), lambda i: (0, i % NBLK, 0)),
    out_shape=jax.ShapeDtypeStruct((2, NP, H), jnp.float32),
    scratch_shapes=[pltpu.VMEM((8, D), jnp.float32)],
    compiler_params=pltpu.CompilerParams(
        dimension_semantics=("arbitrary",)),
)


def _pool_body(x_ref, bat_ref, w1_ref, b1_ref, g_ref, be_ref, w2_ref, b2_ref,
               out_ref, pooled):
    i = pl.program_id(0)

    @pl.when(i == 0)
    def _():
        pooled[...] = jnp.zeros_like(pooled)

    bi = bat_ref[0, 0, :]
    gi = lax.broadcasted_iota(jnp.int32, (NG, BT), 0)
    oh = (gi == bi[None, :]).astype(jnp.float32)
    dot_hi = functools.partial(lax.dot_general,
                               dimension_numbers=(((1,), (0,)), ((), ())),
                               precision=lax.Precision.HIGHEST,
                               preferred_element_type=jnp.float32)
    dot = functools.partial(lax.dot_general,
                            dimension_numbers=(((1,), (0,)), ((), ())),
                            preferred_element_type=jnp.float32)
    pooled[:, 0:H] += dot_hi(oh, x_ref[0])
    pooled[:, H:D] += dot_hi(oh, x_ref[1])

    @pl.when(i == NBLK - 1)
    def _():
        p = pooled[...]
        h = dot(p, w1_ref[...]) + b1_ref[0:1, :]
        mean = jnp.sum(h, axis=0, keepdims=True) / jnp.float32(NG)
        dev = h - mean
        var = jnp.sum(dev * dev, axis=0, keepdims=True) / jnp.float32(NG)
        hn = jnp.maximum(
            g_ref[0:1, :] * dev / jnp.sqrt(var + 1e-5)
            + be_ref[0:1, :], 0.0)
        out_ref[...] = dot(hn, w2_ref[...]) + b2_ref[0:1, :]


_pool = pl.pallas_call(
    _pool_body,
    grid=(NBLK,),
    in_specs=[
        pl.BlockSpec((2, BT, H), lambda i: (0, i, 0)),
        pl.BlockSpec((1, 1, BT), lambda i: (i, 0, 0)),
        pl.BlockSpec((D, D), lambda i: (0, 0)),
        pl.BlockSpec((8, D), lambda i: (0, 0)),
        pl.BlockSpec((8, D), lambda i: (0, 0)),
        pl.BlockSpec((8, D), lambda i: (0, 0)),
        pl.BlockSpec((D, 128), lambda i: (0, 0)),
        pl.BlockSpec((8, 128), lambda i: (0, 0)),
    ],
    out_specs=pl.BlockSpec((NG, 128), lambda i: (0, 0)),
    out_shape=jax.ShapeDtypeStruct((NG, 128), jnp.float32),
    scratch_shapes=[pltpu.VMEM((NG, D), jnp.float32)],
    compiler_params=pltpu.CompilerParams(
        dimension_semantics=("arbitrary",)),
)


def _bcast8(v):
    return jnp.broadcast_to(v.reshape(1, -1).astype(jnp.float32), (8, v.shape[-1]))


def kernel(atom_feats, edge_index, bond_feats, batch, atom_emb, bond_emb, eps,
           conv_W1, conv_b1, conv_W2, conv_b2, bn_gamma, bn_beta,
           ro_W1, ro_b1, ro_bn_gamma, ro_bn_beta, ro_W2, ro_b2):
    i32 = jnp.int32
    f32 = jnp.float32

    af = atom_feats.astype(i32)
    aidx = (af + jnp.arange(9, dtype=i32)[None, :] * 64).T
    aidx = jnp.pad(aidx, ((0, 0), (0, NP - N))).reshape(1, 9 * NPB, 128)
    aidx = jnp.concatenate([aidx, aidx + 576], 0).reshape(-1, 128)

    nrows = jnp.arange(NP, dtype=i32).reshape(NPB, 128)

    bf = bond_feats.astype(i32)
    cid = bf[:, 0] * 64 + bf[:, 1] * 8 + bf[:, 2]
    cid = jnp.pad(cid, (0, EP - E)).reshape(1, EPB, 128)
    cid = jnp.concatenate([cid, cid + NG], 0).reshape(-1, 128)

    src = jnp.pad(edge_index[0].astype(i32), (0, EP - E)).reshape(1, EPB, 128)
    src = jnp.concatenate([src, src + NP], 0).reshape(-1, 128)
    dst = jnp.pad(edge_index[1].astype(i32), (0, EP - E),
                  constant_values=DUMP).reshape(EPB, 128)

    at = atom_emb.astype(f32).reshape(9 * 64, D)
    at2 = jnp.concatenate([at[:, :H], at[:, H:]], axis=0)

    batp = jnp.pad(batch.astype(i32), (0, NP - N),
                   constant_values=NG).reshape(NBLK, 1, BT)

    ct = _combo(bond_emb.astype(f32)).reshape(NC * NG, H)

    x = _encode()(at2, aidx, nrows)

    layer_sc = _layer_sc()
    for l in range(NLAYERS):
        aggr = layer_sc(x, ct, src, dst, cid)
        h2, ssum = _mlp(
            (1.0 + eps[l]).astype(f32).reshape(1),
            x.reshape(2, NP, H), aggr.reshape(2, NP, H),
            conv_W1[l].astype(f32), _bcast8(conv_b1[l]),
            conv_W2[l].astype(f32), _bcast8(conv_b2[l]))
        x = _bnres(h2, x.reshape(2, NP, H), ssum,
                   _bcast8(bn_gamma[l]), _bcast8(bn_beta[l]))
        x = x.reshape(NC * NP, H)

    w2p = jnp.pad(ro_W2.astype(f32), ((0, 0), (0, 128 - T)))
    b2p = _bcast8(jnp.pad(ro_b2.astype(f32), (0, 128 - T)))
    out = _pool(x.reshape(2, NP, H), batp,
                ro_W1.astype(f32), _bcast8(ro_b1),
                _bcast8(ro_bn_gamma), _bcast8(ro_bn_beta), w2p, b2p)
    return out[:, :T]
